# baseline (speedup 1.0000x reference)
import functools

import jax
import jax.numpy as jnp
from jax import lax
from jax.experimental import pallas as pl
from jax.experimental.pallas import tpu as pltpu
from jax.experimental.pallas import tpu_sc as plsc

F32 = jnp.float32
H = 128
EPS = 1e-5
CH = 80
NS = 16
NC = 2

def _mesh():
    return plsc.VectorSubcoreMesh(core_axis_name="c", subcore_axis_name="s")


def _down_body(x_ref, w_ref, b_ref, o_ref):
    y = jnp.dot(x_ref[...], w_ref[...], preferred_element_type=F32) + b_ref[...]
    o_ref[...] = jnp.maximum(y, 0.0)


def _down(x, w, b, tile):
    n, d = x.shape
    return pl.pallas_call(
        _down_body,
        grid=(n // tile,),
        in_specs=[
            pl.BlockSpec((tile, d), lambda i: (i, 0)),
            pl.BlockSpec(w.shape, lambda i: (0, 0)),
            pl.BlockSpec((1, H), lambda i: (0, 0)),
        ],
        out_specs=pl.BlockSpec((tile, H), lambda i: (i, 0)),
        out_shape=jax.ShapeDtypeStruct((n, H), F32),
    )(x, w, b.reshape(1, H))


def _edge_body(fuse_down, rel_ref, tfs_ref, tfo_ref, lng_ref, lnb_ref, gw_ref,
               gb_ref, fw_ref, fb_ref, dw_ref, db_ref,
               reln_ref, ms_ref, mo_ref):
    relin = rel_ref[...]
    if fuse_down:
        relin = jnp.maximum(
            jnp.dot(relin, dw_ref[...], preferred_element_type=F32)
            + db_ref[...], 0.0)
    tfs = tfs_ref[...]
    tfo = tfo_ref[...]

    def norm_pair(u, p):
        mu = 0.5 * (jnp.mean(u, axis=1, keepdims=True)
                    + jnp.mean(p, axis=1, keepdims=True))
        du = u - mu
        dp = p - mu
        var = 0.5 * (jnp.mean(du * du, axis=1, keepdims=True)
                     + jnp.mean(dp * dp, axis=1, keepdims=True))
        inv = lax.rsqrt(var + EPS)
        return du * inv, dp * inv

    def gate(xu, xp, i):
        hu = jnp.maximum(xu * lng_ref[i, :H] + lnb_ref[i, :H], 0.0)
        hp = jnp.maximum(xp * lng_ref[i, H:] + lnb_ref[i, H:], 0.0)
        logits = (jnp.dot(hu, gw_ref[i, :H, :], preferred_element_type=F32)
                  + jnp.dot(hp, gw_ref[i, H:, :], preferred_element_type=F32)
                  + gb_ref[i])
        return jnp.mean(jax.nn.sigmoid(logits), axis=1, keepdims=True)

    xs, xr1 = norm_pair(tfs, relin)
    xo, xr2 = norm_pair(tfo, relin)
    g_p2s = gate(xs, xr1, 0)
    g_p2o = gate(xo, xr2, 1)
    g_s2p = gate(xr1, xs, 2)
    g_o2p = gate(xr2, xo, 3)
    ms_ref[...] = relin * g_p2s
    mo_ref[...] = relin * g_p2o
    gru = 0.5 * (tfs * g_s2p + tfo * g_o2p)
    reln_ref[...] = relin + (
        jnp.dot(jnp.maximum(gru, 0.0), fw_ref[0], preferred_element_type=F32)
        + fb_ref[0]
        + jnp.dot(jnp.maximum(relin, 0.0), fw_ref[1], preferred_element_type=F32)
        + fb_ref[1])


ETILE = 1280


def _edge(rel, tfs, tfo, lng, lnb, gw, gb, fw, fb, dw, db, fuse_down,
          blk0=0, tile=ETILE):
    d = rel.shape[1]
    e = tfs.shape[0]
    full = lambda a: pl.BlockSpec(a.shape, lambda i: (0,) * a.ndim)
    blk = pl.BlockSpec((tile, H), lambda i: (i, 0))
    out_sh = jax.ShapeDtypeStruct((e, H), F32)
    return pl.pallas_call(
        functools.partial(_edge_body, fuse_down),
        grid=(e // tile,),
        in_specs=[pl.BlockSpec((tile, d), lambda i: (blk0 + i, 0)), blk, blk,
                  full(lng), full(lnb), full(gw), full(gb), full(fw), full(fb),
                  full(dw), full(db)],
        out_specs=[blk, blk, blk],
        out_shape=[out_sh, out_sh, out_sh],
    )(rel, tfs, tfo, lng, lnb, gw, gb, fw, fb, dw, db)


def _node_body(nchunk, inst_ref, *rest):
    agg_refs = rest[:2 * nchunk]
    cs_ref, co_ref, fw_ref, fb_ref, o_ref = rest[2 * nchunk:]
    inst = inst_ref[...]
    cs = cs_ref[...][:, :1]
    co = co_ref[...][:, :1]
    asum = agg_refs[0][...]
    aobj = agg_refs[nchunk][...]
    for c in range(1, nchunk):
        asum = asum + agg_refs[c][...]
        aobj = aobj + agg_refs[nchunk + c][...]
    osub = jnp.where(cs > 0.0, asum / jnp.maximum(cs, 1.0), 0.0)
    oobj = jnp.where(co > 0.0, aobj / jnp.maximum(co, 1.0), 0.0)
    gru = 0.5 * (osub + oobj)
    o_ref[...] = inst + (
        jnp.dot(jnp.maximum(gru, 0.0), fw_ref[0], preferred_element_type=F32)
        + fb_ref[0]
        + jnp.dot(jnp.maximum(inst, 0.0), fw_ref[1], preferred_element_type=F32)
        + fb_ref[1])


def _node(inst, agg_s_list, agg_o_list, cnt_s, cnt_o, fw, fb, tile=2000):
    n = inst.shape[0]
    nchunk = len(agg_s_list)
    full = lambda a: pl.BlockSpec(a.shape, lambda i: (0,) * a.ndim)
    blk = pl.BlockSpec((tile, H), lambda i: (i, 0))
    return pl.pallas_call(
        functools.partial(_node_body, nchunk),
        grid=(n // tile,),
        in_specs=[blk] + [blk] * (2 * nchunk) + [blk, blk, full(fw), full(fb)],
        out_specs=blk,
        out_shape=jax.ShapeDtypeStruct((n, H), F32),
    )(inst, *agg_s_list, *agg_o_list, cnt_s, cnt_o, fw, fb)


def _scatter_pair(ms, mo, sub4, obj4, z128):
    e, h = ms.shape
    n = z128.shape[0]
    _, rpt, _, _ = sub4.shape
    nrt = n // NS

    @functools.partial(
        pl.kernel,
        out_type=[jax.ShapeDtypeStruct((n, h), F32),
                  jax.ShapeDtypeStruct((n, h), F32)],
        mesh=_mesh(),
        scratch_types=[
            pltpu.VMEM((1, 1, CH), jnp.int32),
            pltpu.VMEM((1, 1, CH), jnp.int32),
            pltpu.VMEM((CH, h), F32),
            pltpu.VMEM((CH, h), F32),
            pltpu.VMEM_SHARED((n, h), F32),
            pltpu.SemaphoreType.DMA,
            pltpu.SemaphoreType.DMA,
            pltpu.SemaphoreType.DMA,
            pltpu.SemaphoreType.DMA,
        ],
    )
    def k(ms_hbm, mo_hbm, sidx, oidx, z128_hbm,
          as_hbm, ao_hbm,
          idx_a, idx_b, buf_a, buf_b, acc_sh,
          sem_la, sem_lb, sem_ia, sem_ib):
        c = lax.axis_index("c")
        s = lax.axis_index("s")
        pltpu.sync_copy(z128_hbm.at[pl.ds(s * nrt, nrt)],
                        acc_sh.at[pl.ds(s * nrt, nrt)])
        plsc.subcore_barrier()
        row0 = s * rpt

        def run(msg_hbm, idx_hbm):
            pltpu.async_copy(idx_hbm.at[s].at[pl.ds(0, 1)], idx_a, sem_ia)
            pltpu.async_copy(
                msg_hbm.at[pl.ds(row0 * CH, CH)], buf_a, sem_la)

            @pl.loop(0, rpt, step=2)
            def _(i):
                pltpu.async_copy(
                    idx_hbm.at[s].at[pl.ds(i + 1, 1)], idx_b, sem_ib)
                pltpu.async_copy(
                    msg_hbm.at[pl.ds((row0 + i + 1) * CH, CH)], buf_b, sem_lb)
                pltpu.make_async_copy(
                    idx_hbm.at[s].at[pl.ds(i, 1)], idx_a, sem_ia).wait()
                pltpu.make_async_copy(
                    msg_hbm.at[pl.ds((row0 + i) * CH, CH)], buf_a,
                    sem_la).wait()
                pltpu.sync_copy(buf_a, acc_sh.at[idx_a.at[0].at[0]], add=True)

                @pl.when(i + 2 < rpt)
                def _():
                    pltpu.async_copy(
                        idx_hbm.at[s].at[pl.ds(i + 2, 1)], idx_a, sem_ia)
                    pltpu.async_copy(
                        msg_hbm.at[pl.ds((row0 + i + 2) * CH, CH)], buf_a,
                        sem_la)

                pltpu.make_async_copy(
                    idx_hbm.at[s].at[pl.ds(i + 1, 1)], idx_b, sem_ib).wait()
                pltpu.make_async_copy(
                    msg_hbm.at[pl.ds((row0 + i + 1) * CH, CH)], buf_b,
                    sem_lb).wait()
                pltpu.sync_copy(buf_b, acc_sh.at[idx_b.at[0].at[0]], add=True)

        @pl.when(c == 0)
        def _():
            run(ms_hbm, sidx)

        @pl.when(c == 1)
        def _():
            run(mo_hbm, oidx)

        plsc.subcore_barrier()

        @pl.when(c == 0)
        def _():
            pltpu.sync_copy(acc_sh.at[pl.ds(s * nrt, nrt)],
                            as_hbm.at[pl.ds(s * nrt, nrt)])

        @pl.when(c == 1)
        def _():
            pltpu.sync_copy(acc_sh.at[pl.ds(s * nrt, nrt)],
                            ao_hbm.at[pl.ds(s * nrt, nrt)])

    return k(ms, mo, sub4, obj4, z128)


def _counts_pair(sub3, obj3, z128):
    npad, h = z128.shape
    _, rpt, _ = sub3.shape
    nrt = npad // NS

    @functools.partial(
        pl.kernel,
        out_type=[jax.ShapeDtypeStruct((npad, h), F32),
                  jax.ShapeDtypeStruct((npad, h), F32)],
        mesh=_mesh(),
        scratch_types=[
            pltpu.VMEM((rpt, CH), jnp.int32),
            pltpu.VMEM((CH, h), F32),
            pltpu.VMEM_SHARED((npad, h), F32),
        ],
    )
    def k(sidx, oidx, z128_hbm, cs_hbm, co_hbm, idx_v, ones_v, acc_sh):
        c = lax.axis_index("c")
        s = lax.axis_index("s")
        pltpu.sync_copy(z128_hbm.at[pl.ds(s * nrt, nrt)],
                        acc_sh.at[pl.ds(s * nrt, nrt)])
        ones = jnp.ones((16,), F32)

        @pl.loop(0, CH)
        def _(r):
            for j in range(h // 16):
                ones_v[r, pl.ds(j * 16, 16)] = ones

        plsc.subcore_barrier()

        @pl.when(c == 0)
        def _():
            pltpu.sync_copy(sidx.at[s], idx_v)

        @pl.when(c == 1)
        def _():
            pltpu.sync_copy(oidx.at[s], idx_v)

        @pl.loop(0, rpt)
        def _(i):
            pltpu.sync_copy(ones_v, acc_sh.at[idx_v.at[i]], add=True)

        plsc.subcore_barrier()

        @pl.when(c == 0)
        def _():
            pltpu.sync_copy(acc_sh.at[pl.ds(s * nrt, nrt)],
                            cs_hbm.at[pl.ds(s * nrt, nrt)])

        @pl.when(c == 1)
        def _():
            pltpu.sync_copy(acc_sh.at[pl.ds(s * nrt, nrt)],
                            co_hbm.at[pl.ds(s * nrt, nrt)])

    return k(sub3, obj3, z128)


def _gather_min(table, sub3, obj3):
    n, h = table.shape
    _, rpt, _ = sub3.shape
    e = NS * rpt * CH

    @functools.partial(
        pl.kernel,
        out_type=[jax.ShapeDtypeStruct((e, h), F32),
                  jax.ShapeDtypeStruct((e, h), F32)],
        mesh=_mesh(),
        scratch_types=[
            pltpu.VMEM((rpt, CH), jnp.int32),
            pltpu.VMEM((CH, h), F32),
            pltpu.VMEM((CH, h), F32),
            pltpu.SemaphoreType.DMA,
            pltpu.SemaphoreType.DMA,
            pltpu.SemaphoreType.DMA,
            pltpu.SemaphoreType.DMA,
        ],
    )
    def k(tbl, sidx, oidx, outs, outo, idx_v, buf_a, buf_b,
          sem_ga, sem_gb, sem_wa, sem_wb):
        c = lax.axis_index("c")
        s = lax.axis_index("s")
        row0 = s * rpt

        def run(idx_hbm, out_hbm):
            pltpu.sync_copy(idx_hbm.at[s], idx_v)
            pltpu.async_copy(tbl.at[idx_v.at[0]], buf_a, sem_ga)

            @pl.loop(0, rpt, step=2)
            def _(i):
                pltpu.async_copy(tbl.at[idx_v.at[i + 1]], buf_b, sem_gb)
                pltpu.make_async_copy(tbl.at[idx_v.at[i]], buf_a, sem_ga).wait()
                w_a = pltpu.async_copy(
                    buf_a, out_hbm.at[pl.ds((row0 + i) * CH, CH)], sem_wa)
                pltpu.make_async_copy(
                    tbl.at[idx_v.at[i + 1]], buf_b, sem_gb).wait()
                w_b = pltpu.async_copy(
                    buf_b, out_hbm.at[pl.ds((row0 + i + 1) * CH, CH)], sem_wb)
                w_a.wait()

                @pl.when(i + 2 < rpt)
                def _():
                    pltpu.async_copy(tbl.at[idx_v.at[i + 2]], buf_a, sem_ga)

                w_b.wait()

        @pl.when(c == 0)
        def _():
            run(sidx, outs)

        @pl.when(c == 1)
        def _():
            run(oidx, outo)

    return k(table, sub3, obj3)


def kernel(inst_features, rel_features, params, rel_pair_idxs):
    n = inst_features.shape[0]
    e = rel_features.shape[0]
    n_pad = ((n + 128 * NS - 1) // (128 * NS)) * (128 * NS)
    rpt = e // (NS * CH)
    sub3 = rel_pair_idxs[:, 0].reshape(NS, rpt, CH)
    obj3 = rel_pair_idxs[:, 1].reshape(NS, rpt, CH)
    nchunk = 5
    ec = e // nchunk
    rpt_c = ec // (NS * CH)
    sub3c = [rel_pair_idxs[c * ec:(c + 1) * ec, 0].reshape(NS, rpt_c, CH)
             for c in range(nchunk)]
    obj3c = [rel_pair_idxs[c * ec:(c + 1) * ec, 1].reshape(NS, rpt_c, CH)
             for c in range(nchunk)]
    sub4c = [a.reshape(NS, rpt_c, 1, CH) for a in sub3c]
    obj4c = [a.reshape(NS, rpt_c, 1, CH) for a in obj3c]
    z128 = jnp.zeros((n_pad, H), F32)

    gorder = ("gate_pred2sub", "gate_pred2obj", "gate_sub2pred", "gate_obj2pred")
    lng = jnp.stack([params[g]["ln_g"] for g in gorder])
    lnb = jnp.stack([params[g]["ln_b"] for g in gorder])
    gw = jnp.stack([params[g]["W"] for g in gorder])
    gb = jnp.stack([params[g]["b"] for g in gorder])
    fw_rel = jnp.stack([params["pred_fusion"]["Wih"], params["pred_fusion"]["Whh"]])
    fb_rel = jnp.stack([params["pred_fusion"]["bih"], params["pred_fusion"]["bhh"]])
    fw_obj = jnp.stack([params["obj_fusion"]["Wih"], params["obj_fusion"]["Whh"]])
    fb_obj = jnp.stack([params["obj_fusion"]["bih"], params["obj_fusion"]["bhh"]])

    inst = _down(inst_features, params["obj_down_W"], params["obj_down_b"], 2000)
    cnt_s = cnt_o = None
    blk_per_chunk = ec // ETILE
    rel_chunks = [rel_features] * nchunk
    rel_off = [c * blk_per_chunk for c in range(nchunk)]
    for it in range(2):
        agg_s, agg_o = [], []
        new_rel = []
        for c in range(nchunk):
            tfs, tfo = _gather_min(inst, sub3c[c], obj3c[c])
            if cnt_s is None:
                cnt_s, cnt_o = _counts_pair(sub3, obj3, z128)
            reln_c, ms, mo = _edge(rel_chunks[c], tfs, tfo, lng, lnb, gw, gb,
                                   fw_rel, fb_rel,
                                   params["rel_down_W"], params["rel_down_b"],
                                   fuse_down=(it == 0), blk0=rel_off[c])
            a_s, a_o = _scatter_pair(ms, mo, sub4c[c], obj4c[c], z128)
            agg_s.append(a_s)
            agg_o.append(a_o)
            new_rel.append(reln_c)
        inst = _node(inst, agg_s, agg_o, cnt_s, cnt_o, fw_obj, fb_obj)
        rel_chunks = new_rel
        rel_off = [0] * nchunk
    return inst, jnp.concatenate(rel_chunks, axis=0)


# speedup vs baseline: 1.0225x; 1.0225x over previous
import functools

import jax
import jax.numpy as jnp
from jax import lax
from jax.experimental import pallas as pl
from jax.experimental.pallas import tpu as pltpu
from jax.experimental.pallas import tpu_sc as plsc

F32 = jnp.float32
H = 128
EPS = 1e-5
CH = 80
NS = 16
NC = 2

def _mesh():
    return plsc.VectorSubcoreMesh(core_axis_name="c", subcore_axis_name="s")


def _down_body(x_ref, w_ref, b_ref, o_ref):
    y = jnp.dot(x_ref[...], w_ref[...], preferred_element_type=F32) + b_ref[...]
    o_ref[...] = jnp.maximum(y, 0.0)


def _down(x, w, b, tile):
    n, d = x.shape
    return pl.pallas_call(
        _down_body,
        grid=(n // tile,),
        in_specs=[
            pl.BlockSpec((tile, d), lambda i: (i, 0)),
            pl.BlockSpec(w.shape, lambda i: (0, 0)),
            pl.BlockSpec((1, H), lambda i: (0, 0)),
        ],
        out_specs=pl.BlockSpec((tile, H), lambda i: (i, 0)),
        out_shape=jax.ShapeDtypeStruct((n, H), F32),
    )(x, w, b.reshape(1, H))


def _edge_body(fuse_down, rel_ref, tfs_ref, tfo_ref, lng_ref, lnb_ref, gw_ref,
               gb_ref, fw_ref, fb_ref, dw_ref, db_ref,
               reln_ref, ms_ref, mo_ref):
    relin = rel_ref[...]
    if fuse_down:
        relin = jnp.maximum(
            jnp.dot(relin, dw_ref[...], preferred_element_type=F32)
            + db_ref[...], 0.0)
    tfs = tfs_ref[...]
    tfo = tfo_ref[...]

    def norm_pair(u, p):
        mu = 0.5 * (jnp.mean(u, axis=1, keepdims=True)
                    + jnp.mean(p, axis=1, keepdims=True))
        du = u - mu
        dp = p - mu
        var = 0.5 * (jnp.mean(du * du, axis=1, keepdims=True)
                     + jnp.mean(dp * dp, axis=1, keepdims=True))
        inv = lax.rsqrt(var + EPS)
        return du * inv, dp * inv

    def gate(xu, xp, i):
        hu = jnp.maximum(xu * lng_ref[i, :H] + lnb_ref[i, :H], 0.0)
        hp = jnp.maximum(xp * lng_ref[i, H:] + lnb_ref[i, H:], 0.0)
        logits = (jnp.dot(hu, gw_ref[i, :H, :], preferred_element_type=F32)
                  + jnp.dot(hp, gw_ref[i, H:, :], preferred_element_type=F32)
                  + gb_ref[i])
        return jnp.mean(jax.nn.sigmoid(logits), axis=1, keepdims=True)

    xs, xr1 = norm_pair(tfs, relin)
    xo, xr2 = norm_pair(tfo, relin)
    g_p2s = gate(xs, xr1, 0)
    g_p2o = gate(xo, xr2, 1)
    g_s2p = gate(xr1, xs, 2)
    g_o2p = gate(xr2, xo, 3)
    ms_ref[...] = relin * g_p2s
    mo_ref[...] = relin * g_p2o
    gru = 0.5 * (tfs * g_s2p + tfo * g_o2p)
    reln_ref[...] = relin + (
        jnp.dot(jnp.maximum(gru, 0.0), fw_ref[0], preferred_element_type=F32)
        + fb_ref[0]
        + jnp.dot(jnp.maximum(relin, 0.0), fw_ref[1], preferred_element_type=F32)
        + fb_ref[1])


ETILE = 1280


def _edge(rel, tfs, tfo, lng, lnb, gw, gb, fw, fb, dw, db, fuse_down,
          blk0=0, tile=ETILE):
    d = rel.shape[1]
    e = tfs.shape[0]
    full = lambda a: pl.BlockSpec(a.shape, lambda i: (0,) * a.ndim)
    blk = pl.BlockSpec((tile, H), lambda i: (i, 0))
    out_sh = jax.ShapeDtypeStruct((e, H), F32)
    return pl.pallas_call(
        functools.partial(_edge_body, fuse_down),
        grid=(e // tile,),
        in_specs=[pl.BlockSpec((tile, d), lambda i: (blk0 + i, 0)), blk, blk,
                  full(lng), full(lnb), full(gw), full(gb), full(fw), full(fb),
                  full(dw), full(db)],
        out_specs=[blk, blk, blk],
        out_shape=[out_sh, out_sh, out_sh],
    )(rel, tfs, tfo, lng, lnb, gw, gb, fw, fb, dw, db)


def _node_body(nchunk, inst_ref, *rest):
    agg_refs = rest[:2 * nchunk]
    cs_ref, co_ref, fw_ref, fb_ref, o_ref = rest[2 * nchunk:]
    inst = inst_ref[...]
    cs = cs_ref[...][:, :1]
    co = co_ref[...][:, :1]
    asum = agg_refs[0][...]
    aobj = agg_refs[nchunk][...]
    for c in range(1, nchunk):
        asum = asum + agg_refs[c][...]
        aobj = aobj + agg_refs[nchunk + c][...]
    osub = jnp.where(cs > 0.0, asum / jnp.maximum(cs, 1.0), 0.0)
    oobj = jnp.where(co > 0.0, aobj / jnp.maximum(co, 1.0), 0.0)
    gru = 0.5 * (osub + oobj)
    o_ref[...] = inst + (
        jnp.dot(jnp.maximum(gru, 0.0), fw_ref[0], preferred_element_type=F32)
        + fb_ref[0]
        + jnp.dot(jnp.maximum(inst, 0.0), fw_ref[1], preferred_element_type=F32)
        + fb_ref[1])


def _node(inst, agg_s_list, agg_o_list, cnt_s, cnt_o, fw, fb, tile=2000):
    n = inst.shape[0]
    nchunk = len(agg_s_list)
    full = lambda a: pl.BlockSpec(a.shape, lambda i: (0,) * a.ndim)
    blk = pl.BlockSpec((tile, H), lambda i: (i, 0))
    return pl.pallas_call(
        functools.partial(_node_body, nchunk),
        grid=(n // tile,),
        in_specs=[blk] + [blk] * (2 * nchunk) + [blk, blk, full(fw), full(fb)],
        out_specs=blk,
        out_shape=jax.ShapeDtypeStruct((n, H), F32),
    )(inst, *agg_s_list, *agg_o_list, cnt_s, cnt_o, fw, fb)


def _scatter_pair(ms, mo, sub4, obj4, z128):
    e, h = ms.shape
    n = z128.shape[0]
    _, rpt, _, _ = sub4.shape
    nrt = n // NS

    @functools.partial(
        pl.kernel,
        out_type=[jax.ShapeDtypeStruct((n, h), F32),
                  jax.ShapeDtypeStruct((n, h), F32)],
        mesh=_mesh(),
        scratch_types=[
            pltpu.VMEM((1, 1, CH), jnp.int32),
            pltpu.VMEM((1, 1, CH), jnp.int32),
            pltpu.VMEM((CH, h), F32),
            pltpu.VMEM((CH, h), F32),
            pltpu.VMEM_SHARED((n, h), F32),
            pltpu.SemaphoreType.DMA,
            pltpu.SemaphoreType.DMA,
            pltpu.SemaphoreType.DMA,
            pltpu.SemaphoreType.DMA,
        ],
    )
    def k(ms_hbm, mo_hbm, sidx, oidx, z128_hbm,
          as_hbm, ao_hbm,
          idx_a, idx_b, buf_a, buf_b, acc_sh,
          sem_la, sem_lb, sem_ia, sem_ib):
        c = lax.axis_index("c")
        s = lax.axis_index("s")
        pltpu.sync_copy(z128_hbm.at[pl.ds(s * nrt, nrt)],
                        acc_sh.at[pl.ds(s * nrt, nrt)])
        plsc.subcore_barrier()
        row0 = s * rpt

        def run(msg_hbm, idx_hbm):
            pltpu.async_copy(idx_hbm.at[s].at[pl.ds(0, 1)], idx_a, sem_ia)
            pltpu.async_copy(
                msg_hbm.at[pl.ds(row0 * CH, CH)], buf_a, sem_la)

            @pl.loop(0, rpt, step=2)
            def _(i):
                pltpu.async_copy(
                    idx_hbm.at[s].at[pl.ds(i + 1, 1)], idx_b, sem_ib)
                pltpu.async_copy(
                    msg_hbm.at[pl.ds((row0 + i + 1) * CH, CH)], buf_b, sem_lb)
                pltpu.make_async_copy(
                    idx_hbm.at[s].at[pl.ds(i, 1)], idx_a, sem_ia).wait()
                pltpu.make_async_copy(
                    msg_hbm.at[pl.ds((row0 + i) * CH, CH)], buf_a,
                    sem_la).wait()
                pltpu.sync_copy(buf_a, acc_sh.at[idx_a.at[0].at[0]], add=True)

                @pl.when(i + 2 < rpt)
                def _():
                    pltpu.async_copy(
                        idx_hbm.at[s].at[pl.ds(i + 2, 1)], idx_a, sem_ia)
                    pltpu.async_copy(
                        msg_hbm.at[pl.ds((row0 + i + 2) * CH, CH)], buf_a,
                        sem_la)

                pltpu.make_async_copy(
                    idx_hbm.at[s].at[pl.ds(i + 1, 1)], idx_b, sem_ib).wait()
                pltpu.make_async_copy(
                    msg_hbm.at[pl.ds((row0 + i + 1) * CH, CH)], buf_b,
                    sem_lb).wait()
                pltpu.sync_copy(buf_b, acc_sh.at[idx_b.at[0].at[0]], add=True)

        @pl.when(c == 0)
        def _():
            run(ms_hbm, sidx)

        @pl.when(c == 1)
        def _():
            run(mo_hbm, oidx)

        plsc.subcore_barrier()

        @pl.when(c == 0)
        def _():
            pltpu.sync_copy(acc_sh.at[pl.ds(s * nrt, nrt)],
                            as_hbm.at[pl.ds(s * nrt, nrt)])

        @pl.when(c == 1)
        def _():
            pltpu.sync_copy(acc_sh.at[pl.ds(s * nrt, nrt)],
                            ao_hbm.at[pl.ds(s * nrt, nrt)])

    return k(ms, mo, sub4, obj4, z128)


def _counts_pair(sub3, obj3, z128):
    npad, h = z128.shape
    _, rpt, _ = sub3.shape
    nrt = npad // NS

    @functools.partial(
        pl.kernel,
        out_type=[jax.ShapeDtypeStruct((npad, h), F32),
                  jax.ShapeDtypeStruct((npad, h), F32)],
        mesh=_mesh(),
        scratch_types=[
            pltpu.VMEM((rpt, CH), jnp.int32),
            pltpu.VMEM((CH, h), F32),
            pltpu.VMEM_SHARED((npad, h), F32),
        ],
    )
    def k(sidx, oidx, z128_hbm, cs_hbm, co_hbm, idx_v, ones_v, acc_sh):
        c = lax.axis_index("c")
        s = lax.axis_index("s")
        pltpu.sync_copy(z128_hbm.at[pl.ds(s * nrt, nrt)],
                        acc_sh.at[pl.ds(s * nrt, nrt)])
        ones = jnp.ones((16,), F32)

        @pl.loop(0, CH)
        def _(r):
            for j in range(h // 16):
                ones_v[r, pl.ds(j * 16, 16)] = ones

        plsc.subcore_barrier()

        @pl.when(c == 0)
        def _():
            pltpu.sync_copy(sidx.at[s], idx_v)

        @pl.when(c == 1)
        def _():
            pltpu.sync_copy(oidx.at[s], idx_v)

        @pl.loop(0, rpt)
        def _(i):
            pltpu.sync_copy(ones_v, acc_sh.at[idx_v.at[i]], add=True)

        plsc.subcore_barrier()

        @pl.when(c == 0)
        def _():
            pltpu.sync_copy(acc_sh.at[pl.ds(s * nrt, nrt)],
                            cs_hbm.at[pl.ds(s * nrt, nrt)])

        @pl.when(c == 1)
        def _():
            pltpu.sync_copy(acc_sh.at[pl.ds(s * nrt, nrt)],
                            co_hbm.at[pl.ds(s * nrt, nrt)])

    return k(sub3, obj3, z128)


def _gather_min(table, sub3, obj3):
    n, h = table.shape
    _, rpt, _ = sub3.shape
    e = NS * rpt * CH
    nrt = n // NS

    @functools.partial(
        pl.kernel,
        out_type=[jax.ShapeDtypeStruct((e, h), F32),
                  jax.ShapeDtypeStruct((e, h), F32)],
        mesh=_mesh(),
        scratch_types=[
            pltpu.VMEM((rpt, CH), jnp.int32),
            pltpu.VMEM((CH, h), F32),
            pltpu.VMEM((CH, h), F32),
            pltpu.VMEM_SHARED((n, h), F32),
            pltpu.SemaphoreType.DMA,
            pltpu.SemaphoreType.DMA,
            pltpu.SemaphoreType.DMA,
            pltpu.SemaphoreType.DMA,
        ],
    )
    def k(tbl, sidx, oidx, outs, outo, idx_v, buf_a, buf_b, tbl_sh,
          sem_ga, sem_gb, sem_wa, sem_wb):
        c = lax.axis_index("c")
        s = lax.axis_index("s")
        row0 = s * rpt
        pltpu.sync_copy(tbl.at[pl.ds(s * nrt, nrt)],
                        tbl_sh.at[pl.ds(s * nrt, nrt)])
        plsc.subcore_barrier()

        def run(idx_hbm, out_hbm):
            pltpu.sync_copy(idx_hbm.at[s], idx_v)
            pltpu.async_copy(tbl_sh.at[idx_v.at[0]], buf_a, sem_ga)

            @pl.loop(0, rpt, step=2)
            def _(i):
                pltpu.async_copy(tbl_sh.at[idx_v.at[i + 1]], buf_b, sem_gb)
                pltpu.make_async_copy(
                    tbl_sh.at[idx_v.at[i]], buf_a, sem_ga).wait()
                w_a = pltpu.async_copy(
                    buf_a, out_hbm.at[pl.ds((row0 + i) * CH, CH)], sem_wa)
                pltpu.make_async_copy(
                    tbl_sh.at[idx_v.at[i + 1]], buf_b, sem_gb).wait()
                w_b = pltpu.async_copy(
                    buf_b, out_hbm.at[pl.ds((row0 + i + 1) * CH, CH)], sem_wb)
                w_a.wait()

                @pl.when(i + 2 < rpt)
                def _():
                    pltpu.async_copy(tbl_sh.at[idx_v.at[i + 2]], buf_a, sem_ga)

                w_b.wait()

        @pl.when(c == 0)
        def _():
            run(sidx, outs)

        @pl.when(c == 1)
        def _():
            run(oidx, outo)

    return k(table, sub3, obj3)


def kernel(inst_features, rel_features, params, rel_pair_idxs):
    n = inst_features.shape[0]
    e = rel_features.shape[0]
    n_pad = ((n + 128 * NS - 1) // (128 * NS)) * (128 * NS)
    rpt = e // (NS * CH)
    sub3 = rel_pair_idxs[:, 0].reshape(NS, rpt, CH)
    obj3 = rel_pair_idxs[:, 1].reshape(NS, rpt, CH)
    nchunk = 5
    ec = e // nchunk
    rpt_c = ec // (NS * CH)
    sub3c = [rel_pair_idxs[c * ec:(c + 1) * ec, 0].reshape(NS, rpt_c, CH)
             for c in range(nchunk)]
    obj3c = [rel_pair_idxs[c * ec:(c + 1) * ec, 1].reshape(NS, rpt_c, CH)
             for c in range(nchunk)]
    sub4c = [a.reshape(NS, rpt_c, 1, CH) for a in sub3c]
    obj4c = [a.reshape(NS, rpt_c, 1, CH) for a in obj3c]
    z128 = jnp.zeros((n_pad, H), F32)

    gorder = ("gate_pred2sub", "gate_pred2obj", "gate_sub2pred", "gate_obj2pred")
    lng = jnp.stack([params[g]["ln_g"] for g in gorder])
    lnb = jnp.stack([params[g]["ln_b"] for g in gorder])
    gw = jnp.stack([params[g]["W"] for g in gorder])
    gb = jnp.stack([params[g]["b"] for g in gorder])
    fw_rel = jnp.stack([params["pred_fusion"]["Wih"], params["pred_fusion"]["Whh"]])
    fb_rel = jnp.stack([params["pred_fusion"]["bih"], params["pred_fusion"]["bhh"]])
    fw_obj = jnp.stack([params["obj_fusion"]["Wih"], params["obj_fusion"]["Whh"]])
    fb_obj = jnp.stack([params["obj_fusion"]["bih"], params["obj_fusion"]["bhh"]])

    inst = _down(inst_features, params["obj_down_W"], params["obj_down_b"], 2000)
    cnt_s = cnt_o = None
    blk_per_chunk = ec // ETILE
    rel_chunks = [rel_features] * nchunk
    rel_off = [c * blk_per_chunk for c in range(nchunk)]
    for it in range(2):
        agg_s, agg_o = [], []
        new_rel = []
        inst_pad = jnp.concatenate(
            [inst, jnp.zeros((n_pad - n, H), F32)], axis=0)
        for c in range(nchunk):
            tfs, tfo = _gather_min(inst_pad, sub3c[c], obj3c[c])
            if cnt_s is None:
                cnt_s, cnt_o = _counts_pair(sub3, obj3, z128)
            reln_c, ms, mo = _edge(rel_chunks[c], tfs, tfo, lng, lnb, gw, gb,
                                   fw_rel, fb_rel,
                                   params["rel_down_W"], params["rel_down_b"],
                                   fuse_down=(it == 0), blk0=rel_off[c])
            a_s, a_o = _scatter_pair(ms, mo, sub4c[c], obj4c[c], z128)
            agg_s.append(a_s)
            agg_o.append(a_o)
            new_rel.append(reln_c)
        inst = _node(inst, agg_s, agg_o, cnt_s, cnt_o, fw_obj, fb_obj)
        rel_chunks = new_rel
        rel_off = [0] * nchunk
    return inst, jnp.concatenate(rel_chunks, axis=0)
